# baseline (speedup 1.0000x reference)
import jax
import jax.numpy as jnp
from jax.experimental import pallas as pl
from jax.experimental.pallas import tpu as pltpu

BT = 1024
BK = 1024


def _fused_kernel(x_ref, w1_ref, b1_ref, w2_ref, b2_ref, w3_ref, b3_ref,
                  logits_ref, logp_ref, acc_ref, *, nk):
    k = pl.program_id(1)
    part = jnp.dot(x_ref[...].astype(jnp.bfloat16),
                   w1_ref[...].astype(jnp.bfloat16),
                   preferred_element_type=jnp.float32)

    @pl.when(k == 0)
    def _():
        acc_ref[...] = part

    @pl.when(k != 0)
    def _():
        acc_ref[...] += part

    @pl.when(k == nk - 1)
    def _():
        h1 = jnp.maximum(acc_ref[...] + b1_ref[...], 0.0)
        h2 = jnp.maximum(
            jnp.dot(h1, w2_ref[...], preferred_element_type=jnp.float32)
            + b2_ref[...], 0.0)
        logits = (jnp.dot(h2, w3_ref[...],
                          preferred_element_type=jnp.float32) + b3_ref[...])
        m = jnp.max(logits, axis=-1, keepdims=True)
        lse = (jnp.log(jnp.sum(jnp.exp(logits - m), axis=-1, keepdims=True))
               + m)
        logits_ref[...] = logits
        logp_ref[...] = logits - lse


def kernel(state_tensor, W1, b1, W2, b2, W3, b3):
    import functools
    n, d = state_tensor.shape
    e = W3.shape[1]
    nk = d // BK
    out = pl.pallas_call(
        functools.partial(_fused_kernel, nk=nk),
        grid=(n // BT, nk),
        in_specs=[
            pl.BlockSpec((BT, BK), lambda i, k: (i, k)),
            pl.BlockSpec((BK, 128), lambda i, k: (k, 0)),
            pl.BlockSpec((1, 128), lambda i, k: (0, 0)),
            pl.BlockSpec((128, 64), lambda i, k: (0, 0)),
            pl.BlockSpec((1, 64), lambda i, k: (0, 0)),
            pl.BlockSpec((64, e), lambda i, k: (0, 0)),
            pl.BlockSpec((1, e), lambda i, k: (0, 0)),
        ],
        out_specs=[
            pl.BlockSpec((BT, e), lambda i, k: (i, 0)),
            pl.BlockSpec((BT, e), lambda i, k: (i, 0)),
        ],
        out_shape=[
            jax.ShapeDtypeStruct((n, e), jnp.float32),
            jax.ShapeDtypeStruct((n, e), jnp.float32),
        ],
        scratch_shapes=[pltpu.VMEM((BT, 128), jnp.float32)],
        compiler_params=pltpu.CompilerParams(
            dimension_semantics=("arbitrary", "arbitrary")),
    )(state_tensor, W1, b1.reshape(1, -1), W2, b2.reshape(1, -1),
      W3, b3.reshape(1, -1))
    return out[0], out[1]


# speedup vs baseline: 1.0278x; 1.0278x over previous
import functools

import jax
import jax.numpy as jnp
from jax.experimental import pallas as pl
from jax.experimental.pallas import tpu as pltpu

BT = 1024
BK = 1024
XBUF = 6


def _fused_kernel(x_ref, w1_ref, b1_ref, w2_ref, b2_ref, w3_ref, b3_ref,
                  logits_ref, logp_ref, acc_ref, *, nk):
    k = pl.program_id(1)
    w1b = w1_ref[pl.ds(k * BK, BK), :].astype(jnp.bfloat16)
    part = jnp.dot(x_ref[...].astype(jnp.bfloat16), w1b,
                   preferred_element_type=jnp.float32)

    @pl.when(k == 0)
    def _():
        acc_ref[...] = part

    @pl.when(k != 0)
    def _():
        acc_ref[...] += part

    @pl.when(k == nk - 1)
    def _():
        h1 = jnp.maximum(acc_ref[...] + b1_ref[...], 0.0)
        h2 = jnp.maximum(
            jnp.dot(h1, w2_ref[...], preferred_element_type=jnp.float32)
            + b2_ref[...], 0.0)
        logits = (jnp.dot(h2, w3_ref[...],
                          preferred_element_type=jnp.float32) + b3_ref[...])
        m = jnp.max(logits, axis=-1, keepdims=True)
        lse = (jnp.log(jnp.sum(jnp.exp(logits - m), axis=-1, keepdims=True))
               + m)
        logits_ref[...] = logits
        logp_ref[...] = logits - lse


def kernel(state_tensor, W1, b1, W2, b2, W3, b3):
    n, d = state_tensor.shape
    e = W3.shape[1]
    nk = d // BK
    out = pl.pallas_call(
        functools.partial(_fused_kernel, nk=nk),
        grid=(n // BT, nk),
        in_specs=[
            pl.BlockSpec((BT, BK), lambda i, k: (i, k)),
            pl.BlockSpec((d, 128), lambda i, k: (0, 0)),
            pl.BlockSpec((1, 128), lambda i, k: (0, 0)),
            pl.BlockSpec((128, 64), lambda i, k: (0, 0)),
            pl.BlockSpec((1, 64), lambda i, k: (0, 0)),
            pl.BlockSpec((64, e), lambda i, k: (0, 0)),
            pl.BlockSpec((1, e), lambda i, k: (0, 0)),
        ],
        out_specs=[
            pl.BlockSpec((BT, e), lambda i, k: (i, 0)),
            pl.BlockSpec((BT, e), lambda i, k: (i, 0)),
        ],
        out_shape=[
            jax.ShapeDtypeStruct((n, e), jnp.float32),
            jax.ShapeDtypeStruct((n, e), jnp.float32),
        ],
        scratch_shapes=[pltpu.VMEM((BT, 128), jnp.float32)],
        compiler_params=pltpu.CompilerParams(
            dimension_semantics=("arbitrary", "arbitrary")),
    )(state_tensor, W1, b1.reshape(1, -1), W2, b2.reshape(1, -1),
      W3, b3.reshape(1, -1))
    return out[0], out[1]


# speedup vs baseline: 1.2540x; 1.2200x over previous
import jax
import jax.numpy as jnp
from jax.experimental import pallas as pl
from jax.experimental.pallas import tpu as pltpu

BT = 1024
BK = 1024
XBUF = 5
N_TOK = 8192
D_IN = 4096


def _outer(x_hbm, w1_ref, b1_ref, w2_ref, b2_ref, w3_ref, b3_ref,
           logits_hbm, logp_hbm, acc_ref):
    nk = D_IN // BK

    def body(idx, x_tile, logits_blk, logp_blk):
        _, k = idx
        w1b = w1_ref[pl.ds(k * BK, BK), :].astype(jnp.bfloat16)
        part = jnp.dot(x_tile[...].astype(jnp.bfloat16), w1b,
                       preferred_element_type=jnp.float32)

        @pl.when(k == 0)
        def _():
            acc_ref[...] = part

        @pl.when(k != 0)
        def _():
            acc_ref[...] += part

        @pl.when(k == nk - 1)
        def _():
            h1 = jnp.maximum(acc_ref[...] + b1_ref[...], 0.0)
            h2 = jnp.maximum(
                jnp.dot(h1, w2_ref[...], preferred_element_type=jnp.float32)
                + b2_ref[...], 0.0)
            logits = (jnp.dot(h2, w3_ref[...],
                              preferred_element_type=jnp.float32)
                      + b3_ref[...])
            m = jnp.max(logits, axis=-1, keepdims=True)
            lse = (jnp.log(jnp.sum(jnp.exp(logits - m), axis=-1,
                                   keepdims=True)) + m)
            logits_blk[...] = logits
            logp_blk[...] = logits - lse

    pipeline = pltpu.emit_pipeline(
        body,
        grid=(N_TOK // BT, nk),
        in_specs=[
            pl.BlockSpec((BT, BK), lambda i, k: (i, k),
                         pipeline_mode=pl.Buffered(buffer_count=XBUF)),
        ],
        out_specs=[
            pl.BlockSpec((BT, 64), lambda i, k: (i, 0)),
            pl.BlockSpec((BT, 64), lambda i, k: (i, 0)),
        ],
        _explicit_indices=True,
    )
    pipeline(x_hbm, logits_hbm, logp_hbm)


def kernel(state_tensor, W1, b1, W2, b2, W3, b3):
    n, d = state_tensor.shape
    e = W3.shape[1]
    out = pl.pallas_call(
        _outer,
        in_specs=[
            pl.BlockSpec(memory_space=pl.ANY),
            pl.BlockSpec((d, 128), lambda: (0, 0)),
            pl.BlockSpec((1, 128), lambda: (0, 0)),
            pl.BlockSpec((128, 64), lambda: (0, 0)),
            pl.BlockSpec((1, 64), lambda: (0, 0)),
            pl.BlockSpec((64, e), lambda: (0, 0)),
            pl.BlockSpec((1, e), lambda: (0, 0)),
        ],
        out_specs=[
            pl.BlockSpec(memory_space=pl.ANY),
            pl.BlockSpec(memory_space=pl.ANY),
        ],
        out_shape=[
            jax.ShapeDtypeStruct((n, e), jnp.float32),
            jax.ShapeDtypeStruct((n, e), jnp.float32),
        ],
        scratch_shapes=[pltpu.VMEM((BT, 128), jnp.float32)],
    )(state_tensor, W1, b1.reshape(1, -1), W2, b2.reshape(1, -1),
      W3, b3.reshape(1, -1))
    return out[0], out[1]
